# baseline (speedup 1.0000x reference)
import functools

import jax
import jax.numpy as jnp
from jax import lax
from jax.experimental import pallas as pl
from jax.experimental.pallas import tpu as pltpu
from jax.experimental.pallas import tpu_sc as plsc

N_ROWS = 8192
C = 2048
L = 16
NC = 2
NS = 16
NW = NC * NS

A_SC = 4096
SC_RPW = A_SC // NW
R = 8
SC_CHUNKS = SC_RPW // R
GROUPS = C // L

RB = 256
TC_GRID = (N_ROWS - A_SC) // RB


def _sc_body(x_hbm, p_hbm, out_hbm, p_v,
             in0, in1, out0, out1, isem0, isem1, osem0, osem1):
    cid = lax.axis_index("c")
    sid = lax.axis_index("s")
    wid = sid * NC + cid
    base_row = wid * SC_RPW

    pltpu.sync_copy(p_hbm, p_v)

    ins = (in0, in1)
    outs = (out0, out1)
    isems = (isem0, isem1)
    osems = (osem0, osem1)

    def in_copy(ch, b):
        r0 = base_row + ch * R
        return [pltpu.async_copy(x_hbm.at[r0 + r],
                                 ins[b].at[pl.ds(r * C, C)], isems[b])
                for r in range(R)]

    def out_copy(ch, b):
        r0 = base_row + ch * R
        return [pltpu.async_copy(outs[b].at[pl.ds(r * C, C)],
                                 out_hbm.at[r0 + r], osems[b])
                for r in range(R)]

    def permute(b):
        in_v = ins[b]
        out_v = outs[b]

        @plsc.parallel_loop(0, GROUPS, unroll=4)
        def grp_body(g):
            idx0 = p_v[pl.ds(g * L, L)]
            for r in range(R):
                vals = plsc.load_gather(in_v, [idx0 + r * C])
                out_v[pl.ds(r * C + g * L, L)] = vals

    pending_in = [None, None]
    pending_out = [None, None]
    pending_in[0] = in_copy(0, 0)
    for ch in range(SC_CHUNKS):
        b = ch % 2
        if ch + 1 < SC_CHUNKS:
            pending_in[1 - b] = in_copy(ch + 1, 1 - b)
        for h in pending_in[b]:
            h.wait()
        if pending_out[b] is not None:
            for h in pending_out[b]:
                h.wait()
            pending_out[b] = None
        permute(b)
        pending_out[b] = out_copy(ch, b)
    for b in range(2):
        if pending_out[b] is not None:
            for h in pending_out[b]:
                h.wait()


def _sc_permute(x, p32):
    mesh = plsc.VectorSubcoreMesh(core_axis_name="c", subcore_axis_name="s")
    k = functools.partial(
        pl.kernel,
        out_type=jax.ShapeDtypeStruct((A_SC, C), jnp.float32),
        mesh=mesh,
        scratch_types=[
            pltpu.VMEM((C,), jnp.int32),
            pltpu.VMEM((R * C,), jnp.float32),
            pltpu.VMEM((R * C,), jnp.float32),
            pltpu.VMEM((R * C,), jnp.float32),
            pltpu.VMEM((R * C,), jnp.float32),
            pltpu.SemaphoreType.DMA,
            pltpu.SemaphoreType.DMA,
            pltpu.SemaphoreType.DMA,
            pltpu.SemaphoreType.DMA,
        ],
        compiler_params=pltpu.CompilerParams(needs_layout_passes=False),
    )(_sc_body)
    return k(x, p32)


def _tc_body(p_ref, x_ref, out_ref, P_ref):
    @pl.when(pl.program_id(0) == 0)
    def _():
        pv = p_ref[0, :]
        iota = lax.broadcasted_iota(jnp.int32, (C, C), 0)
        P_ref[...] = (iota == pv[None, :]).astype(jnp.bfloat16)

    hi = x_ref[...].astype(jnp.bfloat16)
    out_ref[...] = jnp.dot(hi, P_ref[...], preferred_element_type=jnp.float32)


def _tc_permute(p2d, x):
    return pl.pallas_call(
        _tc_body,
        grid=(TC_GRID,),
        in_specs=[
            pl.BlockSpec((1, C), lambda i: (0, 0)),
            pl.BlockSpec((RB, C), lambda i: (i + A_SC // RB, 0)),
        ],
        out_specs=pl.BlockSpec((RB, C), lambda i: (i + A_SC // RB, 0)),
        out_shape=jax.ShapeDtypeStruct((N_ROWS, C), jnp.float32),
        scratch_shapes=[pltpu.VMEM((C, C), jnp.bfloat16)],
        compiler_params=pltpu.CompilerParams(
            dimension_semantics=("arbitrary",)),
    )(p2d, x)


@jax.jit
def kernel(x, p):
    p32 = p.astype(jnp.int32)
    sc_out = _sc_permute(x, p32)
    tc_out = _tc_permute(p32.reshape(1, C), x)
    return lax.dynamic_update_slice(tc_out, sc_out, (0, 0))


# speedup vs baseline: 1.0250x; 1.0250x over previous
import functools

import jax
import jax.numpy as jnp
from jax import lax
from jax.experimental import pallas as pl
from jax.experimental.pallas import tpu as pltpu
from jax.experimental.pallas import tpu_sc as plsc

N_ROWS = 8192
C = 2048
L = 16
NC = 2
NS = 16
NW = NC * NS

A_SC = 4608
SC_RPW = A_SC // NW
R = 8
SC_CHUNKS = SC_RPW // R
GROUPS = C // L

RB = 256
TC_GRID = (N_ROWS - A_SC) // RB


def _sc_body(x_hbm, p_hbm, out_hbm, p_v,
             in0, in1, out0, out1, isem0, isem1, osem0, osem1):
    cid = lax.axis_index("c")
    sid = lax.axis_index("s")
    wid = sid * NC + cid
    base_row = wid * SC_RPW

    pltpu.sync_copy(p_hbm, p_v)

    ins = (in0, in1)
    outs = (out0, out1)
    isems = (isem0, isem1)
    osems = (osem0, osem1)

    def in_copy(ch, b):
        r0 = base_row + ch * R
        return [pltpu.async_copy(x_hbm.at[r0 + r],
                                 ins[b].at[pl.ds(r * C, C)], isems[b])
                for r in range(R)]

    def out_copy(ch, b):
        r0 = base_row + ch * R
        return [pltpu.async_copy(outs[b].at[pl.ds(r * C, C)],
                                 out_hbm.at[r0 + r], osems[b])
                for r in range(R)]

    def permute(b):
        in_v = ins[b]
        out_v = outs[b]

        @plsc.parallel_loop(0, GROUPS, unroll=4)
        def grp_body(g):
            idx0 = p_v[pl.ds(g * L, L)]
            for r in range(R):
                vals = plsc.load_gather(in_v, [idx0 + r * C])
                out_v[pl.ds(r * C + g * L, L)] = vals

    pending_in = [None, None]
    pending_out = [None, None]
    pending_in[0] = in_copy(0, 0)
    for ch in range(SC_CHUNKS):
        b = ch % 2
        if ch + 1 < SC_CHUNKS:
            pending_in[1 - b] = in_copy(ch + 1, 1 - b)
        for h in pending_in[b]:
            h.wait()
        if pending_out[b] is not None:
            for h in pending_out[b]:
                h.wait()
            pending_out[b] = None
        permute(b)
        pending_out[b] = out_copy(ch, b)
    for b in range(2):
        if pending_out[b] is not None:
            for h in pending_out[b]:
                h.wait()


def _sc_permute(x, p32):
    mesh = plsc.VectorSubcoreMesh(core_axis_name="c", subcore_axis_name="s")
    k = functools.partial(
        pl.kernel,
        out_type=jax.ShapeDtypeStruct((N_ROWS, C), jnp.float32),
        mesh=mesh,
        scratch_types=[
            pltpu.VMEM((C,), jnp.int32),
            pltpu.VMEM((R * C,), jnp.float32),
            pltpu.VMEM((R * C,), jnp.float32),
            pltpu.VMEM((R * C,), jnp.float32),
            pltpu.VMEM((R * C,), jnp.float32),
            pltpu.SemaphoreType.DMA,
            pltpu.SemaphoreType.DMA,
            pltpu.SemaphoreType.DMA,
            pltpu.SemaphoreType.DMA,
        ],
        compiler_params=pltpu.CompilerParams(needs_layout_passes=False),
    )(_sc_body)
    return k(x, p32)


def _tc_body(p_ref, x_ref, out_ref, P_ref):
    @pl.when(pl.program_id(0) == 0)
    def _():
        pv = p_ref[0, :]
        iota = lax.broadcasted_iota(jnp.int32, (C, C), 0)
        P_ref[...] = (iota == pv[None, :]).astype(jnp.bfloat16)

    hi = x_ref[...].astype(jnp.bfloat16)
    out_ref[...] = jnp.dot(hi, P_ref[...], preferred_element_type=jnp.float32)


def _tc_permute(p2d, x):
    return pl.pallas_call(
        _tc_body,
        grid=(TC_GRID,),
        in_specs=[
            pl.BlockSpec((1, C), lambda i: (0, 0)),
            pl.BlockSpec((RB, C), lambda i: (i + A_SC // RB, 0)),
        ],
        out_specs=pl.BlockSpec((RB, C), lambda i: (i, 0)),
        out_shape=jax.ShapeDtypeStruct((N_ROWS - A_SC, C), jnp.float32),
        scratch_shapes=[pltpu.VMEM((C, C), jnp.bfloat16)],
        compiler_params=pltpu.CompilerParams(
            dimension_semantics=("arbitrary",)),
    )(p2d, x)


@jax.jit
def kernel(x, p):
    p32 = p.astype(jnp.int32)
    sc_out = _sc_permute(x, p32)
    tc_out = _tc_permute(p32.reshape(1, C), x)
    return lax.dynamic_update_slice(sc_out, tc_out, (A_SC, 0))
